# baseline (speedup 1.0000x reference)
import jax
import jax.numpy as jnp
from jax.experimental import pallas as pl

B, N, D, H = 64, 1024, 192, 48
BT = 4096
BB = BT // N
G = (B * N) // BT


def _body(x_ref, lu_ref, mt_ref, w1_ref, b1_ref, w2_ref, b2_ref,
          out_ref, m_ref):
    xb = x_ref[...].reshape(BT, D)
    ht = jax.lax.dot_general(
        w1_ref[...], xb, (((1,), (1,)), ((), ())),
        preferred_element_type=jnp.float32)
    g = jax.nn.gelu(ht + b1_ref[...])
    logit = jax.lax.dot_general(
        w2_ref[...], g, (((1,), (0,)), ((), ())),
        preferred_element_type=jnp.float32) + b2_ref[...]
    s = logit - lu_ref[0]
    m_ref[...] = (s > 0)[None].astype(jnp.int8)
    ones_row = jnp.ones((1, D), jnp.float32)
    smat = jax.lax.dot_general(
        s, ones_row, (((0,), (0,)), ((), ())),
        preferred_element_type=jnp.float32)
    out = jnp.where(smat > 0, mt_ref[...], xb)
    out_ref[...] = out.reshape(BB, N, D)


def kernel(x, mask_token, W1, b1, W2, b2):
    u = jax.random.uniform(jax.random.key(42), (B, N, 1), jnp.float32)
    lu = (jnp.log(u) - jnp.log1p(-u)).reshape(G, 1, BT)
    masked, m8 = pl.pallas_call(
        _body,
        grid=(G,),
        in_specs=[
            pl.BlockSpec((BB, N, D), lambda g: (g, 0, 0)),
            pl.BlockSpec((1, 1, BT), lambda g: (g, 0, 0)),
            pl.BlockSpec((1, D), lambda g: (0, 0)),
            pl.BlockSpec((H, D), lambda g: (0, 0)),
            pl.BlockSpec((H, 1), lambda g: (0, 0)),
            pl.BlockSpec((1, H), lambda g: (0, 0)),
            pl.BlockSpec((1, 1), lambda g: (0, 0)),
        ],
        out_specs=[
            pl.BlockSpec((BB, N, D), lambda g: (g, 0, 0)),
            pl.BlockSpec((1, 1, BT), lambda g: (g, 0, 0)),
        ],
        out_shape=[
            jax.ShapeDtypeStruct((B, N, D), jnp.float32),
            jax.ShapeDtypeStruct((G, 1, BT), jnp.int8),
        ],
    )(x, lu, mask_token.reshape(1, D), W1, b1.reshape(H, 1),
      W2, b2.reshape(1, 1))
    return masked, m8.reshape(B, N).astype(jnp.bool_)


# speedup vs baseline: 2.3959x; 2.3959x over previous
import jax
import jax.numpy as jnp
import numpy as np
from jax.experimental import pallas as pl

B, N, D, H = 64, 1024, 192, 48


def _np_uniform_f32(seed: int, size: int) -> np.ndarray:
    def rotl(x, r):
        return ((x << np.uint32(r)) | (x >> np.uint32(32 - r))).astype(np.uint32)

    k1 = np.uint32(np.uint64(seed) >> np.uint64(32))
    k2 = np.uint32(seed & 0xFFFFFFFF)
    rots = [(13, 15, 26, 6), (17, 29, 16, 24)]
    ks = [k1, k2, np.uint32(k1 ^ k2 ^ np.uint32(0x1BD11BDA))]
    with np.errstate(over="ignore"):
        x0 = np.zeros(size, np.uint32) + ks[0]
        x1 = (np.arange(size, dtype=np.uint32) + ks[1]).astype(np.uint32)
        for i in range(5):
            for r in rots[i % 2]:
                x0 = (x0 + x1).astype(np.uint32)
                x1 = (x0 ^ rotl(x1, r)).astype(np.uint32)
            x0 = (x0 + ks[(i + 1) % 3]).astype(np.uint32)
            x1 = (x1 + ks[(i + 2) % 3] + np.uint32(i + 1)).astype(np.uint32)
    bits = x0 ^ x1
    fb = (bits >> np.uint32(9)) | np.uint32(0x3F800000)
    return np.maximum(np.float32(0.0), fb.view(np.float32) - np.float32(1.0))


_U = _np_uniform_f32(42, B * N)
with np.errstate(divide="ignore"):
    _LU = (np.log(_U) - np.log1p(-_U)).astype(np.float32).reshape(B, 1, N)


def _body(x_ref, lu_ref, mt_ref, w1_ref, b1_ref, w2_ref, b2_ref,
          out_ref, m_ref):
    xb = x_ref[0]
    ht = jnp.dot(w1_ref[...], xb,
                 preferred_element_type=jnp.float32)
    g = jax.nn.gelu(ht + b1_ref[...])
    logit = jnp.dot(w2_ref[...], g,
                    preferred_element_type=jnp.float32) + b2_ref[...]
    s = logit - lu_ref[0]
    m_ref[0] = (s > 0).astype(jnp.int8)
    out_ref[0] = jnp.where(s > 0, mt_ref[...], xb)


def kernel(x, mask_token, W1, b1, W2, b2):
    xt = jnp.transpose(x, (0, 2, 1))
    outt, m8 = pl.pallas_call(
        _body,
        grid=(B,),
        in_specs=[
            pl.BlockSpec((1, D, N), lambda g: (g, 0, 0)),
            pl.BlockSpec((1, 1, N), lambda g: (g, 0, 0)),
            pl.BlockSpec((D, 1), lambda g: (0, 0)),
            pl.BlockSpec((H, D), lambda g: (0, 0)),
            pl.BlockSpec((H, 1), lambda g: (0, 0)),
            pl.BlockSpec((1, H), lambda g: (0, 0)),
            pl.BlockSpec((1, 1), lambda g: (0, 0)),
        ],
        out_specs=[
            pl.BlockSpec((1, D, N), lambda g: (g, 0, 0)),
            pl.BlockSpec((1, 1, N), lambda g: (g, 0, 0)),
        ],
        out_shape=[
            jax.ShapeDtypeStruct((B, D, N), jnp.float32),
            jax.ShapeDtypeStruct((B, 1, N), jnp.int8),
        ],
    )(xt, jnp.asarray(_LU), mask_token.reshape(D, 1), W1,
      b1.reshape(H, 1), W2, b2.reshape(1, 1))
    masked = jnp.transpose(outt, (0, 2, 1))
    return masked, m8.reshape(B, N).astype(jnp.bool_)
